# Initial kernel scaffold; baseline (speedup 1.0000x reference)
import jax
import jax.numpy as jnp
from jax.experimental import pallas as pl


def kernel(x, word_w, freq_w, theta_w):
    raise NotImplementedError("write your pallas kernel here")


# speedup vs baseline: 1.0097x; 1.0097x over previous
import jax
import jax.numpy as jnp
from jax import lax
from jax.experimental import pallas as pl

_TWO_PI = 6.283185307179586
_INV_TWO_PI = 0.15915494309189535
_INV_PIO2 = 0.6366197723675814
_PIO2_1 = 1.5707855224609375
_PIO2_2 = 1.0804334124e-05
_PIO2_3 = 6.0770999344e-11


def _sincos(p):
    kf = jnp.floor(p * _INV_PIO2 + 0.5)
    r = ((p - kf * _PIO2_1) - kf * _PIO2_2) - kf * _PIO2_3
    k = kf.astype(jnp.int32)
    r2 = r * r
    sp = r * (1.0 + r2 * (-1.6666667163e-01 + r2 * (8.3333337680e-03
              + r2 * (-1.9841270114e-04 + r2 * 2.7557314297e-06))))
    cp = 1.0 + r2 * (-0.5 + r2 * (4.1666667908e-02
              + r2 * (-1.3888889225e-03 + r2 * 2.4801587642e-05)))
    q = jnp.bitwise_and(k, 3)
    swap = jnp.bitwise_and(q, 1) == 1
    s1 = jnp.where(swap, cp, sp)
    c1 = jnp.where(swap, sp, cp)
    sneg = jnp.bitwise_and(q, 2) == 2
    cneg = jnp.bitwise_and(q + 1, 2) == 2
    return jnp.where(sneg, -s1, s1), jnp.where(cneg, -c1, c1)


_CB = 8
_HB = 56


def _body(n0, hb, w_ref, f_ref, t_ref, x_ref, or_ref, oi_ref):
    bci = pl.program_id(0)
    hi = pl.program_id(1)
    x = x_ref[...]
    W = x.shape[-1]
    is_y = ((bci // n0) % 2) == 1

    @pl.when(jnp.logical_not(is_y))
    def _():
        w = w_ref[:, :, 0:1]
        f = f_ref[:, :, 0:1]
        t = t_ref[:, :, 0:1]
        t = t - _TWO_PI * jnp.floor(t * _INV_TWO_PI)
        pos = (hi * hb + 1 + lax.broadcasted_iota(
            jnp.int32, (x.shape[0], hb, 1), 1)).astype(jnp.float32)
        s, c = _sincos(pos * f + t)
        or_ref[...] = x + w * c
        oi_ref[...] = jnp.broadcast_to(w * s, x.shape)

    @pl.when(is_y)
    def _():
        w = w_ref[:, :, 1:2]
        f = f_ref[:, :, 1:2]
        t = t_ref[:, :, 1:2]
        t = t - _TWO_PI * jnp.floor(t * _INV_TWO_PI)
        pos = (1 + lax.broadcasted_iota(
            jnp.int32, (x.shape[0], 1, W), 2)).astype(jnp.float32)
        s, c = _sincos(pos * f + t)
        or_ref[...] = x + w * c
        oi_ref[...] = jnp.broadcast_to(w * s, x.shape)


def kernel(x, word_w, freq_w, theta_w):
    B, C, H, W = x.shape
    D = C // 2
    cb, hb = _CB, _HB
    n0 = D // cb
    xr = x.reshape(B * C, H, W)
    wT = word_w.T.reshape(D, 1, 2)
    fT = freq_w.T.reshape(D, 1, 2)
    tT = theta_w.T.reshape(D, 1, 2)

    import functools
    body = functools.partial(_body, n0, hb)
    tab_spec = pl.BlockSpec((cb, 1, 2), lambda i, j: (i % n0, 0, 0))
    re, im = pl.pallas_call(
        body,
        grid=(B * C // cb, H // hb),
        in_specs=[
            tab_spec, tab_spec, tab_spec,
            pl.BlockSpec((cb, hb, W), lambda i, j: (i, j, 0)),
        ],
        out_specs=[
            pl.BlockSpec((cb, hb, W), lambda i, j: (i, j, 0)),
            pl.BlockSpec((cb, hb, W), lambda i, j: (i, j, 0)),
        ],
        out_shape=[
            jax.ShapeDtypeStruct((B * C, H, W), jnp.float32),
            jax.ShapeDtypeStruct((B * C, H, W), jnp.float32),
        ],
    )(wT, fT, tT, xr)
    return lax.complex(re, im).reshape(B, C, H, W)


# speedup vs baseline: 1.7931x; 1.7760x over previous
import functools

import jax
import jax.numpy as jnp
import numpy as np
from jax import lax
from jax.experimental import pallas as pl
from jax.sharding import Mesh, PartitionSpec

_TWO_PI = 6.283185307179586
_INV_TWO_PI = 0.15915494309189535
_INV_PIO2 = 0.6366197723675814
_PIO2_1 = 1.5707855224609375
_PIO2_2 = 1.0804334124e-05
_PIO2_3 = 6.0770999344e-11


def _sincos(p):
    kf = jnp.floor(p * _INV_PIO2 + 0.5)
    r = ((p - kf * _PIO2_1) - kf * _PIO2_2) - kf * _PIO2_3
    k = kf.astype(jnp.int32)
    r2 = r * r
    sp = r * (1.0 + r2 * (-1.6666667163e-01 + r2 * (8.3333337680e-03
              + r2 * (-1.9841270114e-04 + r2 * 2.7557314297e-06))))
    cp = 1.0 + r2 * (-0.5 + r2 * (4.1666667908e-02
              + r2 * (-1.3888889225e-03 + r2 * 2.4801587642e-05)))
    q = jnp.bitwise_and(k, 3)
    swap = jnp.bitwise_and(q, 1) == 1
    s1 = jnp.where(swap, cp, sp)
    c1 = jnp.where(swap, sp, cp)
    sneg = jnp.bitwise_and(q, 2) == 2
    cneg = jnp.bitwise_and(q + 1, 2) == 2
    return jnp.where(sneg, -s1, s1), jnp.where(cneg, -c1, c1)


_CB = 8
_HB = 56


def _body(n0, hb, w_ref, f_ref, t_ref, x_ref, or_ref, oi_ref):
    bci = pl.program_id(0)
    hi = pl.program_id(1)
    x = x_ref[...]
    W = x.shape[-1]
    is_y = ((bci // n0) % 2) == 1

    @pl.when(jnp.logical_not(is_y))
    def _():
        w = w_ref[:, :, 0:1]
        f = f_ref[:, :, 0:1]
        t = t_ref[:, :, 0:1]
        t = t - _TWO_PI * jnp.floor(t * _INV_TWO_PI)
        pos = (hi * hb + 1 + lax.broadcasted_iota(
            jnp.int32, (x.shape[0], hb, 1), 1)).astype(jnp.float32)
        s, c = _sincos(pos * f + t)
        or_ref[...] = x + w * c
        oi_ref[...] = jnp.broadcast_to(w * s, x.shape)

    @pl.when(is_y)
    def _():
        w = w_ref[:, :, 1:2]
        f = f_ref[:, :, 1:2]
        t = t_ref[:, :, 1:2]
        t = t - _TWO_PI * jnp.floor(t * _INV_TWO_PI)
        pos = (1 + lax.broadcasted_iota(
            jnp.int32, (x.shape[0], 1, W), 2)).astype(jnp.float32)
        s, c = _sincos(pos * f + t)
        or_ref[...] = x + w * c
        oi_ref[...] = jnp.broadcast_to(w * s, x.shape)


def _planes(x, word_w, freq_w, theta_w):
    B, C, H, W = x.shape
    D = C // 2
    cb, hb = _CB, _HB
    n0 = D // cb
    xr = x.reshape(B * C, H, W)
    wT = word_w.T.reshape(D, 1, 2)
    fT = freq_w.T.reshape(D, 1, 2)
    tT = theta_w.T.reshape(D, 1, 2)

    body = functools.partial(_body, n0, hb)
    tab_spec = pl.BlockSpec((cb, 1, 2), lambda i, j: (i % n0, 0, 0))
    re, im = pl.pallas_call(
        body,
        grid=(B * C // cb, H // hb),
        in_specs=[
            tab_spec, tab_spec, tab_spec,
            pl.BlockSpec((cb, hb, W), lambda i, j: (i, j, 0)),
        ],
        out_specs=[
            pl.BlockSpec((cb, hb, W), lambda i, j: (i, j, 0)),
            pl.BlockSpec((cb, hb, W), lambda i, j: (i, j, 0)),
        ],
        out_shape=[
            jax.ShapeDtypeStruct((B * C, H, W), jnp.float32),
            jax.ShapeDtypeStruct((B * C, H, W), jnp.float32),
        ],
    )(wT, fT, tT, xr)
    return re.reshape(B, C, H, W), im.reshape(B, C, H, W)


def kernel(x, word_w, freq_w, theta_w):
    B = x.shape[0]
    devs = jax.devices()
    nd = next(n for n in range(min(len(devs), B), 0, -1) if B % n == 0)
    if nd <= 1:
        re, im = _planes(x, word_w, freq_w, theta_w)
        return lax.complex(re, im)

    mesh = Mesh(np.array(devs[:nd]), ("b",))
    pb = PartitionSpec("b")
    pr = PartitionSpec()
    sharded = jax.shard_map(
        _planes, mesh=mesh,
        in_specs=(pb, pr, pr, pr),
        out_specs=(pb, pb),
        check_vma=False,
    )
    re, im = sharded(x, word_w, freq_w, theta_w)
    return lax.complex(re, im)


# speedup vs baseline: 1.8140x; 1.0117x over previous
import functools

import jax
import jax.numpy as jnp
import numpy as np
from jax import lax
from jax.experimental import pallas as pl
from jax.sharding import Mesh, PartitionSpec

_TWO_PI = 6.283185307179586
_INV_TWO_PI = 0.15915494309189535
_INV_PIO2 = 0.6366197723675814
_PIO2_1 = 1.5707855224609375
_PIO2_2 = 1.0804334124e-05
_PIO2_3 = 6.0770999344e-11


def _sincos(p):
    kf = jnp.floor(p * _INV_PIO2 + 0.5)
    r = ((p - kf * _PIO2_1) - kf * _PIO2_2) - kf * _PIO2_3
    k = kf.astype(jnp.int32)
    r2 = r * r
    sp = r * (1.0 + r2 * (-1.6666667163e-01 + r2 * (8.3333337680e-03
              + r2 * (-1.9841270114e-04 + r2 * 2.7557314297e-06))))
    cp = 1.0 + r2 * (-0.5 + r2 * (4.1666667908e-02
              + r2 * (-1.3888889225e-03 + r2 * 2.4801587642e-05)))
    q = jnp.bitwise_and(k, 3)
    swap = jnp.bitwise_and(q, 1) == 1
    s1 = jnp.where(swap, cp, sp)
    c1 = jnp.where(swap, sp, cp)
    sneg = jnp.bitwise_and(q, 2) == 2
    cneg = jnp.bitwise_and(q + 1, 2) == 2
    return jnp.where(sneg, -s1, s1), jnp.where(cneg, -c1, c1)


_CB = 16
_HB = 112


def _body(n0, hb, w_ref, f_ref, t_ref, x_ref, or_ref, oi_ref):
    bci = pl.program_id(0)
    hi = pl.program_id(1)
    x = x_ref[...]
    W = x.shape[-1]
    is_y = ((bci // n0) % 2) == 1

    @pl.when(jnp.logical_not(is_y))
    def _():
        w = w_ref[:, :, 0:1]
        f = f_ref[:, :, 0:1]
        t = t_ref[:, :, 0:1]
        t = t - _TWO_PI * jnp.floor(t * _INV_TWO_PI)
        pos = (hi * hb + 1 + lax.broadcasted_iota(
            jnp.int32, (x.shape[0], hb, 1), 1)).astype(jnp.float32)
        s, c = _sincos(pos * f + t)
        or_ref[...] = x + w * c
        oi_ref[...] = jnp.broadcast_to(w * s, x.shape)

    @pl.when(is_y)
    def _():
        w = w_ref[:, :, 1:2]
        f = f_ref[:, :, 1:2]
        t = t_ref[:, :, 1:2]
        t = t - _TWO_PI * jnp.floor(t * _INV_TWO_PI)
        pos = (1 + lax.broadcasted_iota(
            jnp.int32, (x.shape[0], 1, W), 2)).astype(jnp.float32)
        s, c = _sincos(pos * f + t)
        or_ref[...] = x + w * c
        oi_ref[...] = jnp.broadcast_to(w * s, x.shape)


def _planes(x, word_w, freq_w, theta_w):
    B, C, H, W = x.shape
    D = C // 2
    cb, hb = _CB, _HB
    n0 = D // cb
    xr = x.reshape(B * C, H, W)
    wT = word_w.T.reshape(D, 1, 2)
    fT = freq_w.T.reshape(D, 1, 2)
    tT = theta_w.T.reshape(D, 1, 2)

    body = functools.partial(_body, n0, hb)
    tab_spec = pl.BlockSpec((cb, 1, 2), lambda i, j: (i % n0, 0, 0))
    re, im = pl.pallas_call(
        body,
        grid=(B * C // cb, H // hb),
        in_specs=[
            tab_spec, tab_spec, tab_spec,
            pl.BlockSpec((cb, hb, W), lambda i, j: (i, j, 0)),
        ],
        out_specs=[
            pl.BlockSpec((cb, hb, W), lambda i, j: (i, j, 0)),
            pl.BlockSpec((cb, hb, W), lambda i, j: (i, j, 0)),
        ],
        out_shape=[
            jax.ShapeDtypeStruct((B * C, H, W), jnp.float32),
            jax.ShapeDtypeStruct((B * C, H, W), jnp.float32),
        ],
    )(wT, fT, tT, xr)
    return re.reshape(B, C, H, W), im.reshape(B, C, H, W)


def kernel(x, word_w, freq_w, theta_w):
    B = x.shape[0]
    devs = jax.devices()
    nd = next(n for n in range(min(len(devs), B), 0, -1) if B % n == 0)
    if nd <= 1:
        re, im = _planes(x, word_w, freq_w, theta_w)
        return lax.complex(re, im)

    mesh = Mesh(np.array(devs[:nd]), ("b",))
    pb = PartitionSpec("b")
    pr = PartitionSpec()
    sharded = jax.shard_map(
        _planes, mesh=mesh,
        in_specs=(pb, pr, pr, pr),
        out_specs=(pb, pb),
        check_vma=False,
    )
    re, im = sharded(x, word_w, freq_w, theta_w)
    return lax.complex(re, im)
